# Initial kernel scaffold; baseline (speedup 1.0000x reference)
import jax
import jax.numpy as jnp
from jax.experimental import pallas as pl


def kernel(x, embed):
    raise NotImplementedError("write your pallas kernel here")


# speedup vs baseline: 2.7560x; 2.7560x over previous
import math

import jax
import jax.numpy as jnp
from jax.experimental import pallas as pl


def _scale_copy(e_ref, o_ref):
    o_ref[...] = e_ref[...] * (1.0 / math.sqrt(e_ref.shape[-1]))


def kernel(x, embed):
    s = x.shape[-2]
    d = embed.shape[-1]
    block = 512
    return pl.pallas_call(
        _scale_copy,
        grid=(s // block,),
        in_specs=[pl.BlockSpec((block, d), lambda i: (i, 0))],
        out_specs=pl.BlockSpec((block, d), lambda i: (i, 0)),
        out_shape=jax.ShapeDtypeStruct((s, d), embed.dtype),
    )(embed[:s])
